# Initial kernel scaffold; baseline (speedup 1.0000x reference)
import jax
import jax.numpy as jnp
from jax.experimental import pallas as pl


def kernel(x, orbitals_mf, orbitals_hf):
    raise NotImplementedError("write your pallas kernel here")


# speedup vs baseline: 1.5684x; 1.5684x over previous
import functools

import jax
import jax.numpy as jnp
from jax import lax
from jax.experimental import pallas as pl
from jax.experimental.pallas import tpu as pltpu
from jax.experimental.pallas import tpu_sc as plsc

N_SITES = 256
BATCH = 512
D = 512
L = 16


def _make_kernel():
    info = plsc.get_sparse_core_info()
    nc, ns = info.num_cores, info.num_subcores
    nw = nc * ns
    spw = BATCH // nw
    mesh = plsc.VectorSubcoreMesh(core_axis_name="c", subcore_axis_name="s")

    @functools.partial(
        pl.kernel,
        mesh=mesh,
        compiler_params=pltpu.CompilerParams(needs_layout_passes=False),
        out_type=jax.ShapeDtypeStruct((BATCH, N_SITES, D), jnp.float32),
        scratch_types=[
            pltpu.VMEM((N_SITES,), jnp.int32),
            pltpu.VMEM((N_SITES,), jnp.int32),
            pltpu.VMEM((3 * L,), jnp.int32),
            pltpu.VMEM((N_SITES,), jnp.int32),
            pltpu.VMEM((L, D), jnp.float32),
            pltpu.SemaphoreType.DMA,
        ],
    )
    def k(x_hbm, table_hbm, out_hbm, x_v, csu_v, off_v, idx_v, rows_v, sem):
        wid = lax.axis_index("s") * nc + lax.axis_index("c")
        iota = lax.iota(jnp.int32, L)

        def per_sample(s, carry):
            b = wid * spw + s
            pltpu.sync_copy(x_hbm.at[b], x_v)

            for c in range(N_SITES // L):
                xc = x_v[pl.ds(c * L, L)]
                csu_v[pl.ds(c * L, L)] = jnp.cumsum((xc == 1).astype(jnp.int32))

            counts = plsc.load_gather(csu_v, [iota * L + (L - 1)])
            incl = jnp.cumsum(counts)
            off_v[pl.ds(0, L)] = incl
            tot = plsc.load_gather(off_v, [jnp.full((L,), L - 1, jnp.int32)])
            excl = incl - counts
            off_v[pl.ds(L, L)] = excl
            off_v[pl.ds(2 * L, L)] = tot + iota * L - excl

            for c in range(N_SITES // L):
                xc = x_v[pl.ds(c * L, L)]
                m_up = xc == 1
                m_dn = jnp.logical_not(m_up)
                ids = c * L + iota
                cu = csu_v[pl.ds(c * L, L)]
                cd = (iota + 1) - cu
                uo = plsc.load_gather(off_v, [jnp.full((L,), L + c, jnp.int32)])
                do = plsc.load_gather(off_v,
                                      [jnp.full((L,), 2 * L + c, jnp.int32)])
                pu = jnp.where(m_up, uo + cu - 1, 0)
                pd = jnp.where(m_dn, do + cd - 1, 0)
                plsc.store_scatter(idx_v, [pu], ids, mask=m_up)
                plsc.store_scatter(idx_v, [pd], ids + N_SITES, mask=m_dn)

            for g in range(N_SITES // L):
                ivec = idx_v[pl.ds(g * L, L)]
                ivec = jnp.where(ivec >= 0,
                                 jnp.where(ivec < 2 * N_SITES, ivec, 0), 0)
                pltpu.async_copy(table_hbm.at[ivec], rows_v, sem).wait()
                pltpu.sync_copy(rows_v, out_hbm.at[b, pl.ds(g * L, L)])
            return carry

        lax.fori_loop(0, spw, per_sample, 0)

    return k


_sc_gather = _make_kernel()


@jax.jit
def kernel(x, orbitals_mf, orbitals_hf):
    table = jnp.concatenate((orbitals_mf, orbitals_hf), axis=1)
    return _sc_gather(x, table)


# speedup vs baseline: 2.7219x; 1.7355x over previous
import functools

import jax
import jax.numpy as jnp
from jax import lax
from jax.experimental import pallas as pl
from jax.experimental.pallas import tpu as pltpu
from jax.experimental.pallas import tpu_sc as plsc

N_SITES = 256
BATCH = 512
D = 512
L = 16
NBUF = 8


def _make_kernel():
    info = plsc.get_sparse_core_info()
    nc, ns = info.num_cores, info.num_subcores
    nw = nc * ns
    spw = BATCH // nw
    rows_w = spw * N_SITES
    nchunks = rows_w // L
    nsteps = nchunks // NBUF
    mesh = plsc.VectorSubcoreMesh(core_axis_name="c", subcore_axis_name="s")

    @functools.partial(
        pl.kernel,
        mesh=mesh,
        compiler_params=pltpu.CompilerParams(needs_layout_passes=False),
        out_type=jax.ShapeDtypeStruct((BATCH * N_SITES, D), jnp.float32),
        scratch_types=[
            pltpu.VMEM((spw, N_SITES), jnp.int32),
            pltpu.VMEM((N_SITES,), jnp.int32),
            pltpu.VMEM((3 * L,), jnp.int32),
            pltpu.VMEM((rows_w,), jnp.int32),
        ] + [pltpu.VMEM((L, D), jnp.float32)] * NBUF
          + [pltpu.SemaphoreType.DMA] * (2 * NBUF),
    )
    def k(x_hbm, table_hbm, out_hbm, xa_v, csu_v, off_v, idx_v,
          b0, b1, b2, b3, b4, b5, b6, b7,
          g0, g1, g2, g3, g4, g5, g6, g7,
          s0, s1, s2, s3, s4, s5, s6, s7):
        bufs = (b0, b1, b2, b3, b4, b5, b6, b7)
        gsems = (g0, g1, g2, g3, g4, g5, g6, g7)
        ssems = (s0, s1, s2, s3, s4, s5, s6, s7)
        wid = lax.axis_index("s") * nc + lax.axis_index("c")
        iota = lax.iota(jnp.int32, L)
        pltpu.sync_copy(x_hbm.at[pl.ds(wid * spw, spw)], xa_v)

        def compute(s, carry):
            for c in range(N_SITES // L):
                xc = xa_v[s, pl.ds(c * L, L)]
                csu_v[pl.ds(c * L, L)] = jnp.cumsum((xc == 1).astype(jnp.int32))
            counts = plsc.load_gather(csu_v, [iota * L + (L - 1)])
            incl = jnp.cumsum(counts)
            off_v[pl.ds(0, L)] = incl
            tot = plsc.load_gather(off_v, [jnp.full((L,), L - 1, jnp.int32)])
            excl = incl - counts
            off_v[pl.ds(L, L)] = excl
            off_v[pl.ds(2 * L, L)] = tot + iota * L - excl
            sp = jnp.full((L,), s * N_SITES, jnp.int32)
            for c in range(N_SITES // L):
                xc = xa_v[s, pl.ds(c * L, L)]
                m_up = xc == 1
                m_dn = jnp.logical_not(m_up)
                ids = c * L + iota
                cu = csu_v[pl.ds(c * L, L)]
                cd = (iota + 1) - cu
                uo = plsc.load_gather(off_v, [jnp.full((L,), L + c, jnp.int32)])
                do = plsc.load_gather(off_v,
                                      [jnp.full((L,), 2 * L + c, jnp.int32)])
                pu = jnp.where(m_up, uo + cu - 1, 0)
                pd = jnp.where(m_dn, do + cd - 1, 0)
                plsc.store_scatter(idx_v, [sp + pu], ids, mask=m_up)
                plsc.store_scatter(idx_v, [sp + pd], ids + N_SITES, mask=m_dn)
            return carry

        lax.fori_loop(0, spw, compute, 0)

        out_base = wid * rows_w

        def ivec_for(t):
            iv = idx_v[pl.ds(t * L, L)]
            return jnp.where(iv >= 0,
                             jnp.where(iv < 2 * N_SITES, iv, 0), 0)

        def issue_gather(t, j):
            pltpu.async_copy(table_hbm.at[ivec_for(t)], bufs[j], gsems[j])

        def wait_gather(j):
            pltpu.make_async_copy(table_hbm.at[iota], bufs[j], gsems[j]).wait()

        for j in range(NBUF):
            issue_gather(j, j)

        def step(i, carry):
            t0 = i * NBUF
            for j in range(NBUF):
                wait_gather(j)
                pltpu.async_copy(
                    bufs[j], out_hbm.at[pl.ds(out_base + (t0 + j) * L, L)],
                    ssems[j])
            for j in range(NBUF):
                pltpu.make_async_copy(
                    bufs[j], out_hbm.at[pl.ds(0, L)], ssems[j]).wait()

                @pl.when(i < nsteps - 1)
                def _():
                    issue_gather(t0 + NBUF + j, j)
            return carry

        lax.fori_loop(0, nsteps, step, 0)

    return k


_sc_gather = _make_kernel()


@jax.jit
def kernel(x, orbitals_mf, orbitals_hf):
    table = jnp.concatenate((orbitals_mf, orbitals_hf), axis=1)
    out = _sc_gather(x, table)
    return out.reshape(BATCH, N_SITES, D)


# speedup vs baseline: 2.7801x; 1.0214x over previous
import functools

import jax
import jax.numpy as jnp
from jax import lax
from jax.experimental import pallas as pl
from jax.experimental.pallas import tpu as pltpu
from jax.experimental.pallas import tpu_sc as plsc

N_SITES = 256
BATCH = 512
D = 512
L = 16
NBUF = 8


def _make_kernel():
    info = plsc.get_sparse_core_info()
    nc, ns = info.num_cores, info.num_subcores
    nw = nc * ns
    spw = BATCH // nw
    rows_w = spw * N_SITES
    nchunks = rows_w // L
    nsteps = nchunks // NBUF
    mesh = plsc.VectorSubcoreMesh(core_axis_name="c", subcore_axis_name="s")

    @functools.partial(
        pl.kernel,
        mesh=mesh,
        compiler_params=pltpu.CompilerParams(needs_layout_passes=False),
        out_type=jax.ShapeDtypeStruct((BATCH * N_SITES, D), jnp.float32),
        scratch_types=[
            pltpu.VMEM((spw, N_SITES), jnp.int32),
            pltpu.VMEM((N_SITES,), jnp.int32),
            pltpu.VMEM((3 * L,), jnp.int32),
            pltpu.VMEM((rows_w,), jnp.int32),
        ] + [pltpu.VMEM((L, D), jnp.float32)] * NBUF
          + [pltpu.SemaphoreType.DMA] * (2 * NBUF),
    )
    def k(x_hbm, table_hbm, out_hbm, xa_v, csu_v, off_v, idx_v,
          b0, b1, b2, b3, b4, b5, b6, b7,
          g0, g1, g2, g3, g4, g5, g6, g7,
          s0, s1, s2, s3, s4, s5, s6, s7):
        bufs = (b0, b1, b2, b3, b4, b5, b6, b7)
        gsems = (g0, g1, g2, g3, g4, g5, g6, g7)
        ssems = (s0, s1, s2, s3, s4, s5, s6, s7)
        wid = lax.axis_index("s") * nc + lax.axis_index("c")
        iota = lax.iota(jnp.int32, L)
        pltpu.sync_copy(x_hbm.at[pl.ds(wid * spw, spw)], xa_v)

        def compute(s, carry):
            for c in range(N_SITES // L):
                xc = xa_v[s, pl.ds(c * L, L)]
                csu_v[pl.ds(c * L, L)] = jnp.cumsum((xc == 1).astype(jnp.int32))
            counts = plsc.load_gather(csu_v, [iota * L + (L - 1)])
            incl = jnp.cumsum(counts)
            off_v[pl.ds(0, L)] = incl
            tot = plsc.load_gather(off_v, [jnp.full((L,), L - 1, jnp.int32)])
            excl = incl - counts
            off_v[pl.ds(L, L)] = excl
            off_v[pl.ds(2 * L, L)] = tot + iota * L - excl
            sp = jnp.full((L,), s * N_SITES, jnp.int32)
            for c in range(N_SITES // L):
                xc = xa_v[s, pl.ds(c * L, L)]
                m_up = xc == 1
                m_dn = jnp.logical_not(m_up)
                ids = c * L + iota
                cu = csu_v[pl.ds(c * L, L)]
                cd = (iota + 1) - cu
                uo = plsc.load_gather(off_v, [jnp.full((L,), L + c, jnp.int32)])
                do = plsc.load_gather(off_v,
                                      [jnp.full((L,), 2 * L + c, jnp.int32)])
                pu = jnp.where(m_up, uo + cu - 1, 0)
                pd = jnp.where(m_dn, do + cd - 1, 0)
                plsc.store_scatter(idx_v, [sp + pu], ids, mask=m_up)
                plsc.store_scatter(idx_v, [sp + pd], ids + N_SITES, mask=m_dn)
            return carry

        compute(0, 0)

        out_base = wid * rows_w

        def ivec_for(t):
            iv = idx_v[pl.ds(t * L, L)]
            return jnp.where(iv >= 0,
                             jnp.where(iv < 2 * N_SITES, iv, 0), 0)

        def issue_gather(t, j):
            pltpu.async_copy(table_hbm.at[ivec_for(t)], bufs[j], gsems[j])

        def wait_gather(j):
            pltpu.make_async_copy(table_hbm.at[iota], bufs[j], gsems[j]).wait()

        for j in range(NBUF):
            issue_gather(j, j)

        def step(i, carry):
            t0 = i * NBUF
            for j in range(NBUF):
                wait_gather(j)
                pltpu.async_copy(
                    bufs[j], out_hbm.at[pl.ds(out_base + (t0 + j) * L, L)],
                    ssems[j])

            @pl.when(jnp.logical_and(i % 2 == 0, i < 2 * (spw - 1)))
            def _():
                compute(i // 2 + 1, 0)

            for j in range(NBUF):
                pltpu.make_async_copy(
                    bufs[j], out_hbm.at[pl.ds(0, L)], ssems[j]).wait()

                @pl.when(i < nsteps - 1)
                def _():
                    issue_gather(t0 + NBUF + j, j)
            return carry

        lax.fori_loop(0, nsteps, step, 0)

    return k


_sc_gather = _make_kernel()


@jax.jit
def kernel(x, orbitals_mf, orbitals_hf):
    table = jnp.concatenate((orbitals_mf, orbitals_hf), axis=1)
    out = _sc_gather(x, table)
    return out.reshape(BATCH, N_SITES, D)
